# baseline (speedup 1.0000x reference)
import functools
import jax
import jax.numpy as jnp
from jax.experimental import pallas as pl
from jax.experimental.pallas import tpu as pltpu

N = 50000
K = 8
C_IN = 256
C_OUT = 128
N_OUT = K * N
EPS = 1e-5

_BN = 2000
_NB = N // _BN


def _mm_body(x_ref, w_ref, o_ref):
    o_ref[...] = jnp.dot(x_ref[...], w_ref[0], preferred_element_type=jnp.float32)


def _matmul(x, W):
    return pl.pallas_call(
        _mm_body,
        grid=(_NB, K),
        in_specs=[
            pl.BlockSpec((_BN, C_IN), lambda i, k: (i, 0)),
            pl.BlockSpec((1, C_IN, C_OUT), lambda i, k: (k, 0, 0)),
        ],
        out_specs=pl.BlockSpec((_BN, C_OUT), lambda i, k: (k * _NB + i, 0)),
        out_shape=jax.ShapeDtypeStruct((N_OUT, C_OUT), jnp.float32),
    )(x, W)


_BS = 4000
_NSB = N_OUT // _BS


def _stats_body(o_ref, s_ref, acc):
    @pl.when(pl.program_id(0) == 0)
    def _():
        acc[...] = jnp.zeros_like(acc)

    x = o_ref[...]
    acc[0, :] += jnp.sum(x, axis=0)
    acc[1, :] += jnp.sum(x * x, axis=0)

    @pl.when(pl.program_id(0) == _NSB - 1)
    def _():
        s_ref[...] = acc[...]


def _stats(out):
    return pl.pallas_call(
        _stats_body,
        grid=(_NSB,),
        in_specs=[pl.BlockSpec((_BS, C_OUT), lambda i: (i, 0))],
        out_specs=pl.BlockSpec((2, C_OUT), lambda i: (0, 0)),
        out_shape=jax.ShapeDtypeStruct((2, C_OUT), jnp.float32),
        scratch_shapes=[pltpu.VMEM((2, C_OUT), jnp.float32)],
    )(out)


def _norm_body(o_ref, s_ref, g_ref, b_ref, y_ref):
    ssum = s_ref[0, :]
    ssq = s_ref[1, :]
    inv_n = jnp.float32(1.0 / N_OUT)
    mean = ssum * inv_n
    var = ssq * inv_n - mean * mean
    scale = g_ref[0] * jax.lax.rsqrt(var + EPS)
    shift = b_ref[0] - mean * scale
    h = o_ref[...] * scale[None, :] + shift[None, :]
    y_ref[...] = h * 0.5 * (1.0 + jax.lax.erf(h * jnp.float32(0.7071067811865476)))


def _normalize(out, stats, gamma, beta):
    return pl.pallas_call(
        _norm_body,
        grid=(_NSB,),
        in_specs=[
            pl.BlockSpec((_BS, C_OUT), lambda i: (i, 0)),
            pl.BlockSpec((2, C_OUT), lambda i: (0, 0)),
            pl.BlockSpec((1, C_OUT), lambda i: (0, 0)),
            pl.BlockSpec((1, C_OUT), lambda i: (0, 0)),
        ],
        out_specs=pl.BlockSpec((_BS, C_OUT), lambda i: (i, 0)),
        out_shape=jax.ShapeDtypeStruct((N_OUT, C_OUT), jnp.float32),
    )(out, stats, gamma, beta)


def kernel(x, W, gamma, beta, out_map):
    msg = _matmul(x, W)
    out = jnp.zeros((N_OUT, C_OUT), jnp.float32).at[out_map.reshape(-1)].add(msg)
    st = _stats(out)
    return _normalize(out, st, gamma.reshape(1, C_OUT), beta.reshape(1, C_OUT))
